# Initial kernel scaffold; baseline (speedup 1.0000x reference)
import jax
import jax.numpy as jnp
from jax.experimental import pallas as pl


def kernel(p1, x1, o1, p2, x2, o2, W1, b1, g1, be1, W2, b2):
    raise NotImplementedError("write your pallas kernel here")


# speedup vs baseline: 12.8169x; 12.8169x over previous
import functools

import jax
import jax.numpy as jnp
from jax import lax
from jax.experimental import pallas as pl
from jax.experimental.pallas import tpu as pltpu

N1 = 16384
N2 = 4096
C = 512
K = 5
BLK = 256
NBLK = N1 // BLK
BLK2 = 2048
EPS = 1e-5
FINF = 3.0e38


def _phase1_body(p1_ref, x1_ref, p2t_ref, x2b_ref, w1t_ref, b1_ref,
                 w2t_ref, b2_ref, y1_ref, h2_ref, stats_ref):
    i = pl.program_id(0)

    y1 = jnp.dot(x1_ref[...], w1t_ref[...],
                 preferred_element_type=jnp.float32) + b1_ref[...]
    y1_ref[...] = y1

    @pl.when(i == 0)
    def _():
        stats_ref[...] = jnp.zeros_like(stats_ref)

    s1 = jnp.sum(y1, axis=0)
    s2 = jnp.sum(y1 * y1, axis=0)
    stats_ref[...] += jnp.stack([s1, s2])

    p1b = p1_ref[...]
    p2t = p2t_ref[...]
    rn1 = jnp.sum(p1b * p1b, axis=1, keepdims=True)
    rn2 = jnp.sum(p2t * p2t, axis=0, keepdims=True)
    pp_sel = jnp.dot(p1b.astype(jnp.bfloat16), p2t.astype(jnp.bfloat16),
                     preferred_element_type=jnp.float32)
    pp_acc = jnp.dot(p1b, p2t, preferred_element_type=jnp.float32,
                     precision=lax.Precision.HIGHEST)
    d2_sel = rn1 - 2.0 * pp_sel + rn2
    d2_acc = rn1 - 2.0 * pp_acc + rn2

    work = d2_sel
    for _ in range(K - 1):
        m = jnp.min(work, axis=1, keepdims=True)
        work = jnp.where(work <= m, FINF, work)
    thr = jnp.min(work, axis=1, keepdims=True)

    w = jnp.where(d2_sel <= thr, 1.0 / jnp.maximum(d2_acc, 1e-10), 0.0)
    w = w / jnp.sum(w, axis=1, keepdims=True)

    interp = jnp.dot(w.astype(jnp.bfloat16), x2b_ref[...],
                     preferred_element_type=jnp.float32)
    h2 = jnp.dot(interp, w2t_ref[...],
                 preferred_element_type=jnp.float32) + b2_ref[...]
    h2_ref[...] = jnp.maximum(h2, 0.0)


def _phase2_body(y1_ref, h2_ref, stats_ref, g1_ref, be1_ref, out_ref):
    stats = stats_ref[...]
    mean = stats[0:1, :] * (1.0 / N1)
    var = stats[1:2, :] * (1.0 / N1) - mean * mean
    scale = g1_ref[...] * lax.rsqrt(var + EPS)
    h1 = jnp.maximum((y1_ref[...] - mean) * scale + be1_ref[...], 0.0)
    out_ref[...] = h1 + h2_ref[...]


def kernel(p1, x1, o1, p2, x2, o2, W1, b1, g1, be1, W2, b2):
    p2t = p2.T
    x2b = x2.astype(jnp.bfloat16)
    w1t = W1.T
    w2t = W2.T
    b1r = b1.reshape(1, C)
    b2r = b2.reshape(1, C)
    g1r = g1.reshape(1, C)
    be1r = be1.reshape(1, C)

    y1, h2, stats = pl.pallas_call(
        _phase1_body,
        grid=(NBLK,),
        in_specs=[
            pl.BlockSpec((BLK, 3), lambda i: (i, 0)),
            pl.BlockSpec((BLK, 2 * C), lambda i: (i, 0)),
            pl.BlockSpec((3, N2), lambda i: (0, 0)),
            pl.BlockSpec((N2, C), lambda i: (0, 0)),
            pl.BlockSpec((2 * C, C), lambda i: (0, 0)),
            pl.BlockSpec((1, C), lambda i: (0, 0)),
            pl.BlockSpec((C, C), lambda i: (0, 0)),
            pl.BlockSpec((1, C), lambda i: (0, 0)),
        ],
        out_specs=[
            pl.BlockSpec((BLK, C), lambda i: (i, 0)),
            pl.BlockSpec((BLK, C), lambda i: (i, 0)),
            pl.BlockSpec((2, C), lambda i: (0, 0)),
        ],
        out_shape=[
            jax.ShapeDtypeStruct((N1, C), jnp.float32),
            jax.ShapeDtypeStruct((N1, C), jnp.float32),
            jax.ShapeDtypeStruct((2, C), jnp.float32),
        ],
    )(p1, x1, p2t, x2b, w1t, b1r, w2t, b2r)

    out = pl.pallas_call(
        _phase2_body,
        grid=(N1 // BLK2,),
        in_specs=[
            pl.BlockSpec((BLK2, C), lambda i: (i, 0)),
            pl.BlockSpec((BLK2, C), lambda i: (i, 0)),
            pl.BlockSpec((2, C), lambda i: (0, 0)),
            pl.BlockSpec((1, C), lambda i: (0, 0)),
            pl.BlockSpec((1, C), lambda i: (0, 0)),
        ],
        out_specs=pl.BlockSpec((BLK2, C), lambda i: (i, 0)),
        out_shape=jax.ShapeDtypeStruct((N1, C), jnp.float32),
    )(y1, h2, stats, g1r, be1r)
    return out
